# baseline (speedup 1.0000x reference)
import functools

import jax
import jax.numpy as jnp
from jax.experimental import pallas as pl
from jax.experimental.pallas import tpu as pltpu

B = 262144
RB1 = 2048
NB1 = B // RB1
RB2 = 8192
NB2 = B // RB2


def _mlp_body(x_ref, w1_ref, b1_ref, w2_ref, b2_ref, w3_ref, b3_ref,
              logits_ref, maxs_ref, sums_ref):
    h = jnp.dot(x_ref[...], w1_ref[...], preferred_element_type=jnp.float32)
    h = jnp.maximum(h + b1_ref[...], 0.0)
    h = jnp.tanh(jnp.dot(h, w2_ref[...], preferred_element_type=jnp.float32)
                 + b2_ref[...])
    z = jnp.dot(h, w3_ref[...], preferred_element_type=jnp.float32) + b3_ref[...]
    logits_ref[...] = z
    mb = jnp.max(z)
    sb = jnp.sum(jnp.exp(z - mb))
    maxs_ref[...] = jnp.full((1, 1, 8), mb, jnp.float32)
    sums_ref[...] = jnp.full((1, 1, 8), sb, jnp.float32)


def _norm_body(logits_ref, maxs_ref, sums_ref, out_ref):
    mx = maxs_ref[...]
    m = jnp.max(mx)
    s = jnp.sum(sums_ref[...] * jnp.exp(mx - m)) * 0.125
    out_ref[...] = jnp.exp(logits_ref[...] - m) / s


@jax.jit
def kernel(x, wz1, b1, wz2, b2, wz3, b3):
    full = lambda *_: (0, 0)
    logits, maxs, sums = pl.pallas_call(
        _mlp_body,
        grid=(NB1,),
        in_specs=[
            pl.BlockSpec((RB1, 256), lambda i: (i, 0)),
            pl.BlockSpec((256, 64), full),
            pl.BlockSpec((1, 64), full),
            pl.BlockSpec((64, 16), full),
            pl.BlockSpec((1, 16), full),
            pl.BlockSpec((16, 4), full),
            pl.BlockSpec((1, 4), full),
        ],
        out_specs=[
            pl.BlockSpec((RB1, 4), lambda i: (i, 0)),
            pl.BlockSpec((1, 1, 8), lambda i: (i, 0, 0)),
            pl.BlockSpec((1, 1, 8), lambda i: (i, 0, 0)),
        ],
        out_shape=[
            jax.ShapeDtypeStruct((B, 4), jnp.float32),
            jax.ShapeDtypeStruct((NB1, 1, 8), jnp.float32),
            jax.ShapeDtypeStruct((NB1, 1, 8), jnp.float32),
        ],
        compiler_params=pltpu.CompilerParams(
            dimension_semantics=("parallel",),
        ),
    )(x, wz1, b1, wz2, b2, wz3, b3)

    out = pl.pallas_call(
        _norm_body,
        grid=(NB2,),
        in_specs=[
            pl.BlockSpec((RB2, 4), lambda i: (i, 0)),
            pl.BlockSpec((NB1, 1, 8), lambda i: (0, 0, 0)),
            pl.BlockSpec((NB1, 1, 8), lambda i: (0, 0, 0)),
        ],
        out_specs=pl.BlockSpec((RB2, 4), lambda i: (i, 0)),
        out_shape=jax.ShapeDtypeStruct((B, 4), jnp.float32),
        compiler_params=pltpu.CompilerParams(
            dimension_semantics=("parallel",),
        ),
    )(logits, maxs, sums)
    return out
